# baseline (speedup 1.0000x reference)
import functools

import numpy as np
import jax
import jax.numpy as jnp
from jax import lax
from jax.experimental import pallas as pl
from jax.experimental.pallas import tpu as pltpu
from jax.experimental.pallas import tpu_sc as plsc

B = 32
N = 224 * 224
LANES = 16
ROWLEN = 128
ROWS = N // ROWLEN
SUB = ROWLEN // LANES
MIN_I32 = -2147483648


def _sortable(v):
    return v ^ ((v >> 31) & 0x7FFFFFFF)


def _sc_body(cp_hbm, ct_hbm, out_hbm, a_ref, b_ref, res_ref, hist_ref):
    wid = lax.axis_index("s") * 2 + lax.axis_index("c")
    pltpu.sync_copy(cp_hbm.at[wid], a_ref)
    pltpu.sync_copy(ct_hbm.at[wid], b_ref)

    iota16 = lax.iota(jnp.int32, LANES)
    zero_i = jnp.zeros((LANES,), jnp.int32)

    def p0(i, npos_vec):
        for u in range(SUB):
            ctv = b_ref[i, pl.ds(u * LANES, LANES)]
            v = plsc.bitcast(a_ref[i, pl.ds(u * LANES, LANES)], jnp.int32)
            a_ref[i, pl.ds(u * LANES, LANES)] = plsc.bitcast(
                _sortable(v), jnp.float32)
            npos_vec = npos_vec + jnp.where(ctv > 0.1, 1, 0)
        return npos_vec

    npos_vec = lax.fori_loop(0, ROWS, p0, zero_i)
    num_pos = jnp.sum(npos_vec)
    k = jnp.minimum(3 * num_pos, N - 1)

    ones_i = zero_i + 1

    def lane0(vec):
        return jnp.sum(jnp.where(iota16 == 0, vec, 0))

    def pick(tot_vec, need):
        c = plsc.cumsum(lax.rev(tot_vec, (0,)))
        i0 = plsc.all_reduce_ffs(c >= need) + zero_i
        gstar_vec = 15 - i0
        above = jnp.sum(jnp.where(iota16 > gstar_vec, tot_vec, 0))
        return lane0(gstar_vec), above

    def hist_round(r, prefix, k_rem):
        shift = 24 - 8 * r

        def z(i, _):
            hist_ref[i] = zero_i
            return 0
        lax.fori_loop(0, 256, z, 0)

        def body(i, _):
            for u in range(SUB):
                s = plsc.bitcast(a_ref[i, pl.ds(u * LANES, LANES)], jnp.int32)
                bf = lax.shift_right_logical(s ^ MIN_I32, shift)
                bin_ = bf & 255
                if r == 0:
                    plsc.addupdate_scatter(hist_ref, [bin_, iota16], ones_i)
                else:
                    match = lax.shift_right_logical(bf, 8) == prefix
                    plsc.addupdate_scatter(
                        hist_ref, [bin_, iota16], ones_i, mask=match)
            return 0
        lax.fori_loop(0, ROWS, body, 0)

        gt_vec = zero_i
        for g in range(16):
            acc = zero_i
            for j in range(16):
                acc = acc + hist_ref[g * 16 + j]
            gt_vec = jnp.where(iota16 == g, jnp.sum(acc), gt_vec)
        gs, above_g = pick(gt_vec, k_rem)

        ft_vec = zero_i
        for j in range(16):
            ft_vec = jnp.where(iota16 == j, jnp.sum(hist_ref[gs * 16 + j]),
                               ft_vec)
        js, above_f = pick(ft_vec, k_rem - above_g)

        bstar = gs * 16 + js
        return (prefix << 8) | bstar, k_rem - above_g - above_f

    prefix, k_rem = np.int32(0), k
    for r in range(4):
        prefix, k_rem = hist_round(r, prefix, k_rem)
    t = prefix ^ MIN_I32

    def cpass(i, carry):
        gt_vec, ge_vec = carry
        for u in range(SUB):
            s = plsc.bitcast(a_ref[i, pl.ds(u * LANES, LANES)], jnp.int32)
            gt_vec = gt_vec + jnp.where(s > t, 1, 0)
            ge_vec = ge_vec + jnp.where(s >= t, 1, 0)
        return gt_vec, ge_vec

    gt_vec, ge_vec = lax.fori_loop(0, ROWS, cpass, (zero_i, zero_i))
    cnt_gt = jnp.sum(gt_vec)
    cnt_ge = jnp.sum(ge_vec)
    m = k - cnt_gt
    n_ties = cnt_ge - cnt_gt

    def idx_search(_):
        def count_tie_lt(cand):
            def body(i, cnt_vec):
                for u in range(SUB):
                    s = plsc.bitcast(
                        a_ref[i, pl.ds(u * LANES, LANES)], jnp.int32)
                    idx = i * ROWLEN + u * LANES + iota16
                    cnt_vec = cnt_vec + jnp.where((s == t) & (idx < cand), 1, 0)
                return cnt_vec
            return jnp.sum(lax.fori_loop(0, ROWS, body, zero_i))

        def ibody(it, cut):
            cand = cut | jnp.left_shift(1, 16 - it)
            return jnp.where(count_tie_lt(cand) <= m, cand, cut)

        return lax.fori_loop(0, 17, ibody, np.int32(0))

    cut = lax.cond(n_ties == m, lambda _: np.int32(131072), idx_search, 0)

    def fpass(i, carry):
        sq_vec, keep_vec = carry
        for u in range(SUB):
            s = plsc.bitcast(a_ref[i, pl.ds(u * LANES, LANES)], jnp.int32)
            cpv = plsc.bitcast(_sortable(s), jnp.float32)
            ctv = b_ref[i, pl.ds(u * LANES, LANES)]
            idx = i * ROWLEN + u * LANES + iota16
            keep = (ctv > 0.1) | (s > t) | ((s == t) & (idx < cut))
            d = cpv - ctv
            sq_vec = sq_vec + jnp.where(keep, d * d, 0.0)
            keep_vec = keep_vec + jnp.where(keep, 1, 0)
        return sq_vec, keep_vec

    sq_vec, keep_vec = lax.fori_loop(
        0, ROWS, fpass, (jnp.zeros((LANES,), jnp.float32), zero_i))
    sq_sum = jnp.sum(sq_vec)
    n_keep = jnp.sum(keep_vec).astype(jnp.float32)

    res = jnp.where(iota16 == 0, sq_sum,
                    jnp.where(iota16 == 1, n_keep, 0.0))
    res_ref[...] = res
    pltpu.sync_copy(res_ref, out_hbm.at[wid])


@functools.partial(jax.jit, static_argnums=())
def _sc_partials(cp, ct):
    mesh = plsc.VectorSubcoreMesh(core_axis_name="c", subcore_axis_name="s")
    f = functools.partial(
        pl.kernel,
        mesh=mesh,
        compiler_params=pltpu.CompilerParams(
            needs_layout_passes=False, use_tc_tiling_on_sc=False),
        out_type=jax.ShapeDtypeStruct((B, LANES), jnp.float32),
        scratch_types=[
            pltpu.VMEM((ROWS, ROWLEN), jnp.float32),
            pltpu.VMEM((ROWS, ROWLEN), jnp.float32),
            pltpu.VMEM((LANES,), jnp.float32),
            pltpu.VMEM((256, LANES), jnp.int32),
        ],
    )(_sc_body)
    return f(cp, ct)


def _tc_reduce_body(part_ref, sp_ref, st_ref, out_ref):
    p = part_ref[...]
    lane = lax.broadcasted_iota(jnp.int32, p.shape, 1)
    sq_sum = jnp.sum(jnp.where(lane == 0, p, 0.0))
    n_keep = jnp.sum(jnp.where(lane == 1, p, 0.0))
    court = sq_sum / jnp.maximum(n_keep, 1.0)
    d = sp_ref[...] - st_ref[...]
    score = jnp.sum(d * d) / float(B * 8)
    out_ref[0, 0] = court + score


def _tc_reduce(partials, sp, st):
    return pl.pallas_call(
        _tc_reduce_body,
        out_shape=jax.ShapeDtypeStruct((1, 1), jnp.float32),
        out_specs=pl.BlockSpec(memory_space=pltpu.SMEM),
    )(partials, sp, st)


def kernel(court_preds, score_preds, court_targs, score_targs):
    cp = court_preds.reshape(B, ROWS, ROWLEN)
    ct = court_targs.reshape(B, ROWS, ROWLEN)
    partials = _sc_partials(cp, ct)
    out = _tc_reduce(partials, score_preds, score_targs)
    return out[0, 0]


# speedup vs baseline: 1.3230x; 1.3230x over previous
import functools

import numpy as np
import jax
import jax.numpy as jnp
from jax import lax
from jax.experimental import pallas as pl
from jax.experimental.pallas import tpu as pltpu
from jax.experimental.pallas import tpu_sc as plsc

B = 32
N = 224 * 224
LANES = 16
ROWLEN = 128
ROWS = N // ROWLEN
SUB = ROWLEN // LANES
MIN_I32 = -2147483648


def _sortable(v):
    return v ^ ((v >> 31) & 0x7FFFFFFF)


def _sc_body(cp_hbm, ct_hbm, out_hbm, a_ref, b_ref, res_ref):
    wid = lax.axis_index("s") * 2 + lax.axis_index("c")
    pltpu.sync_copy(cp_hbm.at[wid], a_ref)
    pltpu.sync_copy(ct_hbm.at[wid], b_ref.at[pl.ds(0, ROWS)])

    iota16 = lax.iota(jnp.int32, LANES)
    zero_i = jnp.zeros((LANES,), jnp.int32)

    def p0(i, npos_vec):
        for u in range(SUB):
            ctv = b_ref[i, pl.ds(u * LANES, LANES)]
            v = plsc.bitcast(a_ref[i, pl.ds(u * LANES, LANES)], jnp.int32)
            a_ref[i, pl.ds(u * LANES, LANES)] = plsc.bitcast(
                _sortable(v), jnp.float32)
            npos_vec = npos_vec + jnp.where(ctv > 0.1, 1, 0)
        return npos_vec

    npos_vec = lax.fori_loop(0, ROWS, p0, zero_i)
    num_pos = jnp.sum(npos_vec)
    k = jnp.minimum(3 * num_pos, N - 1)

    def count_ge(cand_signed):
        def body(i, cnt_vec):
            for u in range(SUB):
                s = plsc.bitcast(a_ref[i, pl.ds(u * LANES, LANES)], jnp.int32)
                cnt_vec = cnt_vec + jnp.where(s >= cand_signed, 1, 0)
            return cnt_vec
        return jnp.sum(lax.fori_loop(0, ROWS, body, zero_i))

    def sbody(it, t_off):
        cand = t_off | jnp.left_shift(1, 31 - it)
        cnt = count_ge(cand ^ MIN_I32)
        return jnp.where(cnt >= k, cand, t_off)

    t_off8 = lax.fori_loop(0, 8, sbody, np.int32(0))
    p8 = lax.shift_right_logical(t_off8, 24)

    def compact(i, carry):
        cnt_l, above_vec = carry
        for u in range(SUB):
            sv = a_ref[i, pl.ds(u * LANES, LANES)]
            s = plsc.bitcast(sv, jnp.int32)
            bf = lax.shift_right_logical(s ^ MIN_I32, 24)
            match = bf == p8
            row = cnt_l & 511
            col = iota16 * 8 + lax.shift_right_logical(cnt_l, 9)
            plsc.store_scatter(b_ref, [row, col], sv, mask=match)
            cnt_l = cnt_l + jnp.where(match, 1, 0)
            above_vec = above_vec + jnp.where(bf > p8, 1, 0)
        return cnt_l, above_vec

    cnt_l, above_vec = lax.fori_loop(0, ROWS, compact, (zero_i, zero_i))
    above_cls = jnp.sum(above_vec)
    maxc = jnp.max(cnt_l)

    def count_ge_c(cand_signed):
        def body(rr, cnt_vec):
            row = zero_i + (rr & 511)
            col = iota16 * 8 + lax.shift_right_logical(rr, 9)
            sv = plsc.load_gather(b_ref, [row, col])
            s = plsc.bitcast(sv, jnp.int32)
            keep = (rr < cnt_l) & (s >= cand_signed)
            return cnt_vec + jnp.where(keep, 1, 0)
        return jnp.sum(lax.fori_loop(0, maxc, body, zero_i))

    def sbody_c(it, t_off):
        cand = t_off | jnp.left_shift(1, 23 - it)
        cnt = above_cls + count_ge_c(cand ^ MIN_I32)
        return jnp.where(cnt >= k, cand, t_off)

    t_off = lax.fori_loop(0, 24, sbody_c, t_off8)
    t = t_off ^ MIN_I32

    def cpass(rr, carry):
        gt_vec, ge_vec = carry
        row = zero_i + (rr & 511)
        col = iota16 * 8 + lax.shift_right_logical(rr, 9)
        s = plsc.bitcast(plsc.load_gather(b_ref, [row, col]), jnp.int32)
        valid = rr < cnt_l
        gt_vec = gt_vec + jnp.where(valid & (s > t), 1, 0)
        ge_vec = ge_vec + jnp.where(valid & (s >= t), 1, 0)
        return gt_vec, ge_vec

    gt_vec, ge_vec = lax.fori_loop(0, maxc, cpass, (zero_i, zero_i))
    cnt_gt = above_cls + jnp.sum(gt_vec)
    cnt_ge = above_cls + jnp.sum(ge_vec)

    pltpu.sync_copy(ct_hbm.at[wid], b_ref.at[pl.ds(0, ROWS)])
    m = k - cnt_gt
    n_ties = cnt_ge - cnt_gt

    def idx_search(_):
        def count_tie_lt(cand):
            def body(i, cnt_vec):
                for u in range(SUB):
                    s = plsc.bitcast(
                        a_ref[i, pl.ds(u * LANES, LANES)], jnp.int32)
                    idx = i * ROWLEN + u * LANES + iota16
                    cnt_vec = cnt_vec + jnp.where((s == t) & (idx < cand), 1, 0)
                return cnt_vec
            return jnp.sum(lax.fori_loop(0, ROWS, body, zero_i))

        def ibody(it, cut):
            cand = cut | jnp.left_shift(1, 16 - it)
            return jnp.where(count_tie_lt(cand) <= m, cand, cut)

        return lax.fori_loop(0, 17, ibody, np.int32(0))

    cut = lax.cond(n_ties == m, lambda _: np.int32(131072), idx_search, 0)

    def fpass(i, carry):
        sq_vec, keep_vec = carry
        for u in range(SUB):
            s = plsc.bitcast(a_ref[i, pl.ds(u * LANES, LANES)], jnp.int32)
            cpv = plsc.bitcast(_sortable(s), jnp.float32)
            ctv = b_ref[i, pl.ds(u * LANES, LANES)]
            idx = i * ROWLEN + u * LANES + iota16
            keep = (ctv > 0.1) | (s > t) | ((s == t) & (idx < cut))
            d = cpv - ctv
            sq_vec = sq_vec + jnp.where(keep, d * d, 0.0)
            keep_vec = keep_vec + jnp.where(keep, 1, 0)
        return sq_vec, keep_vec

    sq_vec, keep_vec = lax.fori_loop(
        0, ROWS, fpass, (jnp.zeros((LANES,), jnp.float32), zero_i))
    sq_sum = jnp.sum(sq_vec)
    n_keep = jnp.sum(keep_vec).astype(jnp.float32)

    res = jnp.where(iota16 == 0, sq_sum,
                    jnp.where(iota16 == 1, n_keep, 0.0))
    res_ref[...] = res
    pltpu.sync_copy(res_ref, out_hbm.at[wid])


@functools.partial(jax.jit, static_argnums=())
def _sc_partials(cp, ct):
    mesh = plsc.VectorSubcoreMesh(core_axis_name="c", subcore_axis_name="s")
    f = functools.partial(
        pl.kernel,
        mesh=mesh,
        compiler_params=pltpu.CompilerParams(
            needs_layout_passes=False, use_tc_tiling_on_sc=False),
        out_type=jax.ShapeDtypeStruct((B, LANES), jnp.float32),
        scratch_types=[
            pltpu.VMEM((ROWS, ROWLEN), jnp.float32),
            pltpu.VMEM((512, ROWLEN), jnp.float32),
            pltpu.VMEM((LANES,), jnp.float32),
        ],
    )(_sc_body)
    return f(cp, ct)


def _tc_reduce_body(part_ref, sp_ref, st_ref, out_ref):
    p = part_ref[...]
    lane = lax.broadcasted_iota(jnp.int32, p.shape, 1)
    sq_sum = jnp.sum(jnp.where(lane == 0, p, 0.0))
    n_keep = jnp.sum(jnp.where(lane == 1, p, 0.0))
    court = sq_sum / jnp.maximum(n_keep, 1.0)
    d = sp_ref[...] - st_ref[...]
    score = jnp.sum(d * d) / float(B * 8)
    out_ref[0, 0] = court + score


def _tc_reduce(partials, sp, st):
    return pl.pallas_call(
        _tc_reduce_body,
        out_shape=jax.ShapeDtypeStruct((1, 1), jnp.float32),
        out_specs=pl.BlockSpec(memory_space=pltpu.SMEM),
    )(partials, sp, st)


def kernel(court_preds, score_preds, court_targs, score_targs):
    cp = court_preds.reshape(B, ROWS, ROWLEN)
    ct = court_targs.reshape(B, ROWS, ROWLEN)
    partials = _sc_partials(cp, ct)
    out = _tc_reduce(partials, score_preds, score_targs)
    return out[0, 0]


# speedup vs baseline: 1.3492x; 1.0198x over previous
import functools

import numpy as np
import jax
import jax.numpy as jnp
from jax import lax
from jax.experimental import pallas as pl
from jax.experimental.pallas import tpu as pltpu
from jax.experimental.pallas import tpu_sc as plsc

B = 32
N = 224 * 224
LANES = 16
ROWLEN = 128
ROWS = N // ROWLEN
SUB = ROWLEN // LANES
MIN_I32 = -2147483648
CAP = 1792


def _sortable(v):
    return v ^ ((v >> 31) & 0x7FFFFFFF)


def _sc_body(cp_hbm, ct_hbm, out_hbm, a_ref, b_ref, c_ref, res_ref,
             sem1, sem2):
    wid = lax.axis_index("s") * 2 + lax.axis_index("c")
    h1 = pltpu.async_copy(cp_hbm.at[wid], a_ref, sem1)
    h2 = pltpu.async_copy(ct_hbm.at[wid], b_ref, sem2)
    h1.wait()
    h2.wait()

    iota16 = lax.iota(jnp.int32, LANES)
    zero_i = jnp.zeros((LANES,), jnp.int32)

    def p0(i, npos_vec):
        for u in range(SUB):
            ctv = b_ref[i, pl.ds(u * LANES, LANES)]
            v = plsc.bitcast(a_ref[i, pl.ds(u * LANES, LANES)], jnp.int32)
            a_ref[i, pl.ds(u * LANES, LANES)] = plsc.bitcast(
                _sortable(v), jnp.float32)
            npos_vec = npos_vec + jnp.where(ctv > 0.1, 1, 0)
        return npos_vec

    npos_vec = lax.fori_loop(0, ROWS, p0, zero_i)
    num_pos = jnp.sum(npos_vec)
    k = jnp.minimum(3 * num_pos, N - 1)

    def count_ge(cand_signed):
        def body(i, cnt_vec):
            for u in range(SUB):
                s = plsc.bitcast(a_ref[i, pl.ds(u * LANES, LANES)], jnp.int32)
                cnt_vec = cnt_vec + jnp.where(s >= cand_signed, 1, 0)
            return cnt_vec
        return jnp.sum(lax.fori_loop(0, ROWS, body, zero_i))

    def sbody(it, t_off):
        cand = t_off | jnp.left_shift(1, 31 - it)
        cnt = count_ge(cand ^ MIN_I32)
        return jnp.where(cnt >= k, cand, t_off)

    t_off8 = lax.fori_loop(0, 8, sbody, np.int32(0))
    p8 = lax.shift_right_logical(t_off8, 24)

    def compact(i, carry):
        cnt_l, above_vec = carry
        for u in range(SUB):
            sv = a_ref[i, pl.ds(u * LANES, LANES)]
            s = plsc.bitcast(sv, jnp.int32)
            bf = lax.shift_right_logical(s ^ MIN_I32, 24)
            match = bf == p8
            cc = jnp.minimum(cnt_l, CAP - 1)
            row = cc & 255
            col = iota16 * 7 + lax.shift_right_logical(cc, 8)
            plsc.store_scatter(c_ref, [row, col], sv,
                               mask=match & (cnt_l < CAP))
            cnt_l = cnt_l + jnp.where(match, 1, 0)
            above_vec = above_vec + jnp.where(bf > p8, 1, 0)
        return cnt_l, above_vec

    cnt_l, above_vec = lax.fori_loop(0, ROWS, compact, (zero_i, zero_i))
    above_cls = jnp.sum(above_vec)
    maxc = jnp.max(cnt_l)

    def finish_compacted(_):
        def count_ge_c(cand_signed):
            def body(rr, cnt_vec):
                row = zero_i + (rr & 255)
                col = iota16 * 7 + lax.shift_right_logical(rr, 8)
                sv = plsc.load_gather(c_ref, [row, col])
                s = plsc.bitcast(sv, jnp.int32)
                keep = (rr < cnt_l) & (s >= cand_signed)
                return cnt_vec + jnp.where(keep, 1, 0)
            return jnp.sum(lax.fori_loop(0, maxc, body, zero_i))

        def sbody_c(it, t_off):
            cand = t_off | jnp.left_shift(1, 23 - it)
            cnt = above_cls + count_ge_c(cand ^ MIN_I32)
            return jnp.where(cnt >= k, cand, t_off)

        t_off = lax.fori_loop(0, 24, sbody_c, t_off8)
        tt = t_off ^ MIN_I32

        def cpass(rr, carry):
            gt_vec, ge_vec = carry
            row = zero_i + (rr & 255)
            col = iota16 * 7 + lax.shift_right_logical(rr, 8)
            s = plsc.bitcast(plsc.load_gather(c_ref, [row, col]), jnp.int32)
            valid = rr < cnt_l
            gt_vec = gt_vec + jnp.where(valid & (s > tt), 1, 0)
            ge_vec = ge_vec + jnp.where(valid & (s >= tt), 1, 0)
            return gt_vec, ge_vec

        gt_vec, ge_vec = lax.fori_loop(0, maxc, cpass, (zero_i, zero_i))
        return tt, above_cls + jnp.sum(gt_vec), above_cls + jnp.sum(ge_vec)

    def finish_full(_):
        def sbody_f(it, t_off):
            cand = t_off | jnp.left_shift(1, 23 - it)
            cnt = count_ge(cand ^ MIN_I32)
            return jnp.where(cnt >= k, cand, t_off)

        t_off = lax.fori_loop(0, 24, sbody_f, t_off8)
        tt = t_off ^ MIN_I32

        def cpass(i, carry):
            gt_vec, ge_vec = carry
            for u in range(SUB):
                s = plsc.bitcast(a_ref[i, pl.ds(u * LANES, LANES)], jnp.int32)
                gt_vec = gt_vec + jnp.where(s > tt, 1, 0)
                ge_vec = ge_vec + jnp.where(s >= tt, 1, 0)
            return gt_vec, ge_vec

        gt_vec, ge_vec = lax.fori_loop(0, ROWS, cpass, (zero_i, zero_i))
        return tt, jnp.sum(gt_vec), jnp.sum(ge_vec)

    t, cnt_gt, cnt_ge = lax.cond(maxc <= CAP, finish_compacted, finish_full, 0)
    m = k - cnt_gt
    n_ties = cnt_ge - cnt_gt

    def idx_search(_):
        def count_tie_lt(cand):
            def body(i, cnt_vec):
                for u in range(SUB):
                    s = plsc.bitcast(
                        a_ref[i, pl.ds(u * LANES, LANES)], jnp.int32)
                    idx = i * ROWLEN + u * LANES + iota16
                    cnt_vec = cnt_vec + jnp.where((s == t) & (idx < cand), 1, 0)
                return cnt_vec
            return jnp.sum(lax.fori_loop(0, ROWS, body, zero_i))

        def ibody(it, cut):
            cand = cut | jnp.left_shift(1, 16 - it)
            return jnp.where(count_tie_lt(cand) <= m, cand, cut)

        return lax.fori_loop(0, 17, ibody, np.int32(0))

    cut = lax.cond(n_ties == m, lambda _: np.int32(131072), idx_search, 0)

    def fpass(i, carry):
        sq_vec, keep_vec = carry
        for u in range(SUB):
            s = plsc.bitcast(a_ref[i, pl.ds(u * LANES, LANES)], jnp.int32)
            cpv = plsc.bitcast(_sortable(s), jnp.float32)
            ctv = b_ref[i, pl.ds(u * LANES, LANES)]
            idx = i * ROWLEN + u * LANES + iota16
            keep = (ctv > 0.1) | (s > t) | ((s == t) & (idx < cut))
            d = cpv - ctv
            sq_vec = sq_vec + jnp.where(keep, d * d, 0.0)
            keep_vec = keep_vec + jnp.where(keep, 1, 0)
        return sq_vec, keep_vec

    sq_vec, keep_vec = lax.fori_loop(
        0, ROWS, fpass, (jnp.zeros((LANES,), jnp.float32), zero_i))
    sq_sum = jnp.sum(sq_vec)
    n_keep = jnp.sum(keep_vec).astype(jnp.float32)

    res = jnp.where(iota16 == 0, sq_sum,
                    jnp.where(iota16 == 1, n_keep, 0.0))
    res_ref[...] = res
    pltpu.sync_copy(res_ref, out_hbm.at[wid])


@functools.partial(jax.jit, static_argnums=())
def _sc_partials(cp, ct):
    mesh = plsc.VectorSubcoreMesh(core_axis_name="c", subcore_axis_name="s")
    f = functools.partial(
        pl.kernel,
        mesh=mesh,
        compiler_params=pltpu.CompilerParams(
            needs_layout_passes=False, use_tc_tiling_on_sc=False),
        out_type=jax.ShapeDtypeStruct((B, LANES), jnp.float32),
        scratch_types=[
            pltpu.VMEM((ROWS, ROWLEN), jnp.float32),
            pltpu.VMEM((ROWS, ROWLEN), jnp.float32),
            pltpu.VMEM((256, 112), jnp.float32),
            pltpu.VMEM((LANES,), jnp.float32),
            pltpu.SemaphoreType.DMA,
            pltpu.SemaphoreType.DMA,
        ],
    )(_sc_body)
    return f(cp, ct)


def _tc_reduce_body(part_ref, sp_ref, st_ref, out_ref):
    p = part_ref[...]
    lane = lax.broadcasted_iota(jnp.int32, p.shape, 1)
    sq_sum = jnp.sum(jnp.where(lane == 0, p, 0.0))
    n_keep = jnp.sum(jnp.where(lane == 1, p, 0.0))
    court = sq_sum / jnp.maximum(n_keep, 1.0)
    d = sp_ref[...] - st_ref[...]
    score = jnp.sum(d * d) / float(B * 8)
    out_ref[0, 0] = court + score


def _tc_reduce(partials, sp, st):
    return pl.pallas_call(
        _tc_reduce_body,
        out_shape=jax.ShapeDtypeStruct((1, 1), jnp.float32),
        out_specs=pl.BlockSpec(memory_space=pltpu.SMEM),
    )(partials, sp, st)


def kernel(court_preds, score_preds, court_targs, score_targs):
    cp = court_preds.reshape(B, ROWS, ROWLEN)
    ct = court_targs.reshape(B, ROWS, ROWLEN)
    partials = _sc_partials(cp, ct)
    out = _tc_reduce(partials, score_preds, score_targs)
    return out[0, 0]


# speedup vs baseline: 1.6662x; 1.2350x over previous
import functools

import numpy as np
import jax
import jax.numpy as jnp
from jax import lax
from jax.experimental import pallas as pl
from jax.experimental.pallas import tpu as pltpu
from jax.experimental.pallas import tpu_sc as plsc

B = 32
N = 224 * 224
LANES = 16
ROWLEN = 128
ROWS = N // ROWLEN
SUB = ROWLEN // LANES
MIN_I32 = -2147483648
CAP = 1792


def _sortable(v):
    return v ^ ((v >> 31) & 0x7FFFFFFF)


def _sc_body(cp_hbm, ct_hbm, out_hbm, a_ref, b_ref, c_ref, res_ref,
             sem1, sem2):
    wid = lax.axis_index("s") * 2 + lax.axis_index("c")
    h1 = pltpu.async_copy(cp_hbm.at[wid], a_ref, sem1)
    h2 = pltpu.async_copy(ct_hbm.at[wid], b_ref, sem2)
    h1.wait()
    h2.wait()

    iota16 = lax.iota(jnp.int32, LANES)
    zero_i = jnp.zeros((LANES,), jnp.int32)

    def p0(i, npos_vec):
        for u in range(SUB):
            ctv = b_ref[i, pl.ds(u * LANES, LANES)]
            v = plsc.bitcast(a_ref[i, pl.ds(u * LANES, LANES)], jnp.int32)
            a_ref[i, pl.ds(u * LANES, LANES)] = plsc.bitcast(
                _sortable(v), jnp.float32)
            npos_vec = npos_vec + jnp.where(ctv > 0.1, 1, 0)
        return npos_vec

    npos_vec = lax.fori_loop(0, ROWS, p0, zero_i)
    num_pos = jnp.sum(npos_vec)
    k = jnp.minimum(3 * num_pos, N - 1)

    def count_ge(cand_signed):
        def body(i, cnt_vec):
            for u in range(SUB):
                s = plsc.bitcast(a_ref[i, pl.ds(u * LANES, LANES)], jnp.int32)
                cnt_vec = cnt_vec + jnp.where(s >= cand_signed, 1, 0)
            return cnt_vec
        return jnp.sum(lax.fori_loop(0, ROWS, body, zero_i))

    def sbody(it, t_off):
        cand = t_off | jnp.left_shift(1, 31 - it)
        cnt = count_ge(cand ^ MIN_I32)
        return jnp.where(cnt >= k, cand, t_off)

    t_off8 = lax.fori_loop(0, 8, sbody, np.int32(0))
    p8 = lax.shift_right_logical(t_off8, 24)

    def compact(i, carry):
        cnt_l, above_vec = carry
        for u in range(SUB):
            sv = a_ref[i, pl.ds(u * LANES, LANES)]
            s = plsc.bitcast(sv, jnp.int32)
            bf = lax.shift_right_logical(s ^ MIN_I32, 24)
            match = bf == p8
            cc = jnp.minimum(cnt_l, CAP - 1)
            row = cc & 255
            col = iota16 * 7 + lax.shift_right_logical(cc, 8)
            plsc.store_scatter(c_ref, [row, col], sv,
                               mask=match & (cnt_l < CAP))
            cnt_l = cnt_l + jnp.where(match, 1, 0)
            above_vec = above_vec + jnp.where(bf > p8, 1, 0)
        return cnt_l, above_vec

    cnt_l, above_vec = lax.fori_loop(0, ROWS, compact, (zero_i, zero_i))
    above_cls = jnp.sum(above_vec)
    maxc = jnp.max(cnt_l)

    def finish_compacted(_):
        def count_ge_c(cand_signed):
            def body(rr, cnt_vec):
                row = zero_i + (rr & 255)
                col = iota16 * 7 + lax.shift_right_logical(rr, 8)
                sv = plsc.load_gather(c_ref, [row, col])
                s = plsc.bitcast(sv, jnp.int32)
                keep = (rr < cnt_l) & (s >= cand_signed)
                return cnt_vec + jnp.where(keep, 1, 0)
            return jnp.sum(lax.fori_loop(0, maxc, body, zero_i))

        def sbody_c(it, t_off):
            cand = t_off | jnp.left_shift(1, 23 - it)
            cnt = above_cls + count_ge_c(cand ^ MIN_I32)
            return jnp.where(cnt >= k, cand, t_off)

        t_off = lax.fori_loop(0, 24, sbody_c, t_off8)
        tt = t_off ^ MIN_I32

        def cpass(rr, carry):
            gt_vec, ge_vec = carry
            row = zero_i + (rr & 255)
            col = iota16 * 7 + lax.shift_right_logical(rr, 8)
            s = plsc.bitcast(plsc.load_gather(c_ref, [row, col]), jnp.int32)
            valid = rr < cnt_l
            gt_vec = gt_vec + jnp.where(valid & (s > tt), 1, 0)
            ge_vec = ge_vec + jnp.where(valid & (s >= tt), 1, 0)
            return gt_vec, ge_vec

        gt_vec, ge_vec = lax.fori_loop(0, maxc, cpass, (zero_i, zero_i))
        return tt, above_cls + jnp.sum(gt_vec), above_cls + jnp.sum(ge_vec)

    def finish_full(_):
        def sbody_f(it, t_off):
            cand = t_off | jnp.left_shift(1, 23 - it)
            cnt = count_ge(cand ^ MIN_I32)
            return jnp.where(cnt >= k, cand, t_off)

        t_off = lax.fori_loop(0, 24, sbody_f, t_off8)
        tt = t_off ^ MIN_I32

        def cpass(i, carry):
            gt_vec, ge_vec = carry
            for u in range(SUB):
                s = plsc.bitcast(a_ref[i, pl.ds(u * LANES, LANES)], jnp.int32)
                gt_vec = gt_vec + jnp.where(s > tt, 1, 0)
                ge_vec = ge_vec + jnp.where(s >= tt, 1, 0)
            return gt_vec, ge_vec

        gt_vec, ge_vec = lax.fori_loop(0, ROWS, cpass, (zero_i, zero_i))
        return tt, jnp.sum(gt_vec), jnp.sum(ge_vec)

    t, cnt_gt, cnt_ge = lax.cond(maxc <= CAP, finish_compacted, finish_full, 0)
    m = k - cnt_gt
    n_ties = cnt_ge - cnt_gt

    def idx_search(_):
        def count_tie_lt(cand):
            def body(i, cnt_vec):
                for u in range(SUB):
                    s = plsc.bitcast(
                        a_ref[i, pl.ds(u * LANES, LANES)], jnp.int32)
                    idx = i * ROWLEN + u * LANES + iota16
                    cnt_vec = cnt_vec + jnp.where((s == t) & (idx < cand), 1, 0)
                return cnt_vec
            return jnp.sum(lax.fori_loop(0, ROWS, body, zero_i))

        def ibody(it, cut):
            cand = cut | jnp.left_shift(1, 16 - it)
            return jnp.where(count_tie_lt(cand) <= m, cand, cut)

        return lax.fori_loop(0, 17, ibody, np.int32(0))

    cut = lax.cond(n_ties == m, lambda _: np.int32(131072), idx_search, 0)

    def fpass_simple(_):
        def fpass(i, carry):
            sq_vec, keep_vec = carry
            for u in range(SUB):
                s = plsc.bitcast(a_ref[i, pl.ds(u * LANES, LANES)], jnp.int32)
                cpv = plsc.bitcast(_sortable(s), jnp.float32)
                ctv = b_ref[i, pl.ds(u * LANES, LANES)]
                keep = (ctv > 0.1) | (s >= t)
                d = cpv - ctv
                sq_vec = sq_vec + jnp.where(keep, d * d, 0.0)
                keep_vec = keep_vec + jnp.where(keep, 1, 0)
            return sq_vec, keep_vec
        return lax.fori_loop(
            0, ROWS, fpass, (jnp.zeros((LANES,), jnp.float32), zero_i))

    def fpass_ties(_):
        def fpass(i, carry):
            sq_vec, keep_vec = carry
            for u in range(SUB):
                s = plsc.bitcast(a_ref[i, pl.ds(u * LANES, LANES)], jnp.int32)
                cpv = plsc.bitcast(_sortable(s), jnp.float32)
                ctv = b_ref[i, pl.ds(u * LANES, LANES)]
                idx = i * ROWLEN + u * LANES + iota16
                keep = (ctv > 0.1) | (s > t) | ((s == t) & (idx < cut))
                d = cpv - ctv
                sq_vec = sq_vec + jnp.where(keep, d * d, 0.0)
                keep_vec = keep_vec + jnp.where(keep, 1, 0)
            return sq_vec, keep_vec
        return lax.fori_loop(
            0, ROWS, fpass, (jnp.zeros((LANES,), jnp.float32), zero_i))

    sq_vec, keep_vec = lax.cond(cut == 131072, fpass_simple, fpass_ties, 0)
    sq_sum = jnp.sum(sq_vec)
    n_keep = jnp.sum(keep_vec).astype(jnp.float32)

    res = jnp.where(iota16 == 0, sq_sum,
                    jnp.where(iota16 == 1, n_keep, 0.0))
    res_ref[...] = res
    pltpu.sync_copy(res_ref, out_hbm.at[wid])


@functools.partial(jax.jit, static_argnums=())
def _sc_partials(cp, ct):
    mesh = plsc.VectorSubcoreMesh(core_axis_name="c", subcore_axis_name="s")
    f = functools.partial(
        pl.kernel,
        mesh=mesh,
        compiler_params=pltpu.CompilerParams(
            needs_layout_passes=False, use_tc_tiling_on_sc=False),
        out_type=jax.ShapeDtypeStruct((B, LANES), jnp.float32),
        scratch_types=[
            pltpu.VMEM((ROWS, ROWLEN), jnp.float32),
            pltpu.VMEM((ROWS, ROWLEN), jnp.float32),
            pltpu.VMEM((256, 112), jnp.float32),
            pltpu.VMEM((LANES,), jnp.float32),
            pltpu.SemaphoreType.DMA,
            pltpu.SemaphoreType.DMA,
        ],
    )(_sc_body)
    return f(cp, ct)


def _tc_reduce_body(part_ref, sp_ref, st_ref, out_ref):
    p = part_ref[...]
    lane = lax.broadcasted_iota(jnp.int32, p.shape, 1)
    sq_sum = jnp.sum(jnp.where(lane == 0, p, 0.0))
    n_keep = jnp.sum(jnp.where(lane == 1, p, 0.0))
    court = sq_sum / jnp.maximum(n_keep, 1.0)
    d = sp_ref[...] - st_ref[...]
    score = jnp.sum(d * d) / float(B * 8)
    out_ref[0, 0] = court + score


def _tc_reduce(partials, sp, st):
    return pl.pallas_call(
        _tc_reduce_body,
        out_shape=jax.ShapeDtypeStruct((1, 1), jnp.float32),
        out_specs=pl.BlockSpec(memory_space=pltpu.SMEM),
    )(partials, sp, st)


def kernel(court_preds, score_preds, court_targs, score_targs):
    cp = court_preds.reshape(B, ROWS, ROWLEN)
    ct = court_targs.reshape(B, ROWS, ROWLEN)
    partials = _sc_partials(cp, ct)
    out = _tc_reduce(partials, score_preds, score_targs)
    return out[0, 0]


# speedup vs baseline: 1.7447x; 1.0471x over previous
import functools

import numpy as np
import jax
import jax.numpy as jnp
from jax import lax
from jax.experimental import pallas as pl
from jax.experimental.pallas import tpu as pltpu
from jax.experimental.pallas import tpu_sc as plsc

B = 32
N = 224 * 224
LANES = 16
ROWLEN = 128
ROWS = N // ROWLEN
SUB = ROWLEN // LANES
MIN_I32 = -2147483648
CAP = 224


def _sortable(v):
    return v ^ ((v >> 31) & 0x7FFFFFFF)


def _sc_body(cp_hbm, ct_hbm, out_hbm, a_ref, b_ref, c_ref, res_ref,
             sem1, sem2):
    wid = lax.axis_index("s") * 2 + lax.axis_index("c")
    h1 = pltpu.async_copy(cp_hbm.at[wid], a_ref, sem1)
    h2 = pltpu.async_copy(ct_hbm.at[wid], b_ref, sem2)
    h1.wait()
    h2.wait()

    iota16 = lax.iota(jnp.int32, LANES)
    zero_i = jnp.zeros((LANES,), jnp.int32)

    def p0(i, npos_vec):
        for u in range(SUB):
            ctv = b_ref[i, pl.ds(u * LANES, LANES)]
            v = plsc.bitcast(a_ref[i, pl.ds(u * LANES, LANES)], jnp.int32)
            a_ref[i, pl.ds(u * LANES, LANES)] = plsc.bitcast(
                _sortable(v), jnp.float32)
            npos_vec = npos_vec + jnp.where(ctv > 0.1, 1, 0)
        return npos_vec

    npos_vec = lax.fori_loop(0, ROWS, p0, zero_i)
    num_pos = jnp.sum(npos_vec)
    k = jnp.minimum(3 * num_pos, N - 1)

    def count_ge(cand_signed):
        def body(i, cnt_vec):
            for u in range(SUB):
                s = plsc.bitcast(a_ref[i, pl.ds(u * LANES, LANES)], jnp.int32)
                cnt_vec = cnt_vec + jnp.where(s >= cand_signed, 1, 0)
            return cnt_vec
        return jnp.sum(lax.fori_loop(0, ROWS, body, zero_i))

    def sbody(it, t_off):
        cand = t_off | jnp.left_shift(1, 31 - it)
        cnt = count_ge(cand ^ MIN_I32)
        return jnp.where(cnt >= k, cand, t_off)

    t_off8 = lax.fori_loop(0, 8, sbody, np.int32(0))
    p8 = lax.shift_right_logical(t_off8, 24)

    def compact(i, carry):
        cnts = carry[:SUB]
        above_vec = carry[SUB]
        new_cnts = []
        for u in range(SUB):
            cnt_u = cnts[u]
            sv = a_ref[i, pl.ds(u * LANES, LANES)]
            s = plsc.bitcast(sv, jnp.int32)
            bf = lax.shift_right_logical(s ^ MIN_I32, 24)
            match = bf == p8
            row = jnp.minimum(cnt_u, CAP - 1)
            col = iota16 + u * LANES
            plsc.store_scatter(c_ref, [row, col], sv,
                               mask=match & (cnt_u < CAP))
            new_cnts.append(cnt_u + jnp.where(match, 1, 0))
            above_vec = above_vec + jnp.where(bf > p8, 1, 0)
        return tuple(new_cnts) + (above_vec,)

    init = tuple(zero_i for _ in range(SUB)) + (zero_i,)
    res_c = lax.fori_loop(0, ROWS, compact, init)
    cnts = res_c[:SUB]
    above_cls = jnp.sum(res_c[SUB])
    mx = cnts[0]
    for u in range(1, SUB):
        mx = jnp.maximum(mx, cnts[u])
    maxc = jnp.max(mx)

    def finish_compacted(_):
        def count_ge_c(cand_signed):
            def body(rr, cnt_vec):
                for u in range(SUB):
                    sv = c_ref[rr, pl.ds(u * LANES, LANES)]
                    s = plsc.bitcast(sv, jnp.int32)
                    keep = (rr < cnts[u]) & (s >= cand_signed)
                    cnt_vec = cnt_vec + jnp.where(keep, 1, 0)
                return cnt_vec
            return jnp.sum(lax.fori_loop(0, maxc, body, zero_i))

        def sbody_c(it, t_off):
            cand = t_off | jnp.left_shift(1, 23 - it)
            cnt = above_cls + count_ge_c(cand ^ MIN_I32)
            return jnp.where(cnt >= k, cand, t_off)

        t_off = lax.fori_loop(0, 24, sbody_c, t_off8)
        tt = t_off ^ MIN_I32

        def cpass(rr, carry):
            gt_vec, ge_vec = carry
            for u in range(SUB):
                s = plsc.bitcast(c_ref[rr, pl.ds(u * LANES, LANES)],
                                 jnp.int32)
                valid = rr < cnts[u]
                gt_vec = gt_vec + jnp.where(valid & (s > tt), 1, 0)
                ge_vec = ge_vec + jnp.where(valid & (s >= tt), 1, 0)
            return gt_vec, ge_vec

        gt_vec, ge_vec = lax.fori_loop(0, maxc, cpass, (zero_i, zero_i))
        return tt, above_cls + jnp.sum(gt_vec), above_cls + jnp.sum(ge_vec)

    def finish_full(_):
        def sbody_f(it, t_off):
            cand = t_off | jnp.left_shift(1, 23 - it)
            cnt = count_ge(cand ^ MIN_I32)
            return jnp.where(cnt >= k, cand, t_off)

        t_off = lax.fori_loop(0, 24, sbody_f, t_off8)
        tt = t_off ^ MIN_I32

        def cpass(i, carry):
            gt_vec, ge_vec = carry
            for u in range(SUB):
                s = plsc.bitcast(a_ref[i, pl.ds(u * LANES, LANES)], jnp.int32)
                gt_vec = gt_vec + jnp.where(s > tt, 1, 0)
                ge_vec = ge_vec + jnp.where(s >= tt, 1, 0)
            return gt_vec, ge_vec

        gt_vec, ge_vec = lax.fori_loop(0, ROWS, cpass, (zero_i, zero_i))
        return tt, jnp.sum(gt_vec), jnp.sum(ge_vec)

    t, cnt_gt, cnt_ge = lax.cond(maxc <= CAP, finish_compacted, finish_full, 0)
    m = k - cnt_gt
    n_ties = cnt_ge - cnt_gt

    def idx_search(_):
        def count_tie_lt(cand):
            def body(i, cnt_vec):
                for u in range(SUB):
                    s = plsc.bitcast(
                        a_ref[i, pl.ds(u * LANES, LANES)], jnp.int32)
                    idx = i * ROWLEN + u * LANES + iota16
                    cnt_vec = cnt_vec + jnp.where((s == t) & (idx < cand), 1, 0)
                return cnt_vec
            return jnp.sum(lax.fori_loop(0, ROWS, body, zero_i))

        def ibody(it, cut):
            cand = cut | jnp.left_shift(1, 16 - it)
            return jnp.where(count_tie_lt(cand) <= m, cand, cut)

        return lax.fori_loop(0, 17, ibody, np.int32(0))

    cut = lax.cond(n_ties == m, lambda _: np.int32(131072), idx_search, 0)

    def fpass_simple(_):
        def fpass(i, carry):
            sq_vec, keep_vec = carry
            for u in range(SUB):
                s = plsc.bitcast(a_ref[i, pl.ds(u * LANES, LANES)], jnp.int32)
                cpv = plsc.bitcast(_sortable(s), jnp.float32)
                ctv = b_ref[i, pl.ds(u * LANES, LANES)]
                keep = (ctv > 0.1) | (s >= t)
                d = cpv - ctv
                sq_vec = sq_vec + jnp.where(keep, d * d, 0.0)
                keep_vec = keep_vec + jnp.where(keep, 1, 0)
            return sq_vec, keep_vec
        return lax.fori_loop(
            0, ROWS, fpass, (jnp.zeros((LANES,), jnp.float32), zero_i))

    def fpass_ties(_):
        def fpass(i, carry):
            sq_vec, keep_vec = carry
            for u in range(SUB):
                s = plsc.bitcast(a_ref[i, pl.ds(u * LANES, LANES)], jnp.int32)
                cpv = plsc.bitcast(_sortable(s), jnp.float32)
                ctv = b_ref[i, pl.ds(u * LANES, LANES)]
                idx = i * ROWLEN + u * LANES + iota16
                keep = (ctv > 0.1) | (s > t) | ((s == t) & (idx < cut))
                d = cpv - ctv
                sq_vec = sq_vec + jnp.where(keep, d * d, 0.0)
                keep_vec = keep_vec + jnp.where(keep, 1, 0)
            return sq_vec, keep_vec
        return lax.fori_loop(
            0, ROWS, fpass, (jnp.zeros((LANES,), jnp.float32), zero_i))

    sq_vec, keep_vec = lax.cond(cut == 131072, fpass_simple, fpass_ties, 0)
    sq_sum = jnp.sum(sq_vec)
    n_keep = jnp.sum(keep_vec).astype(jnp.float32)

    res = jnp.where(iota16 == 0, sq_sum,
                    jnp.where(iota16 == 1, n_keep, 0.0))
    res_ref[...] = res
    pltpu.sync_copy(res_ref, out_hbm.at[wid])


@functools.partial(jax.jit, static_argnums=())
def _sc_partials(cp, ct):
    mesh = plsc.VectorSubcoreMesh(core_axis_name="c", subcore_axis_name="s")
    f = functools.partial(
        pl.kernel,
        mesh=mesh,
        compiler_params=pltpu.CompilerParams(
            needs_layout_passes=False, use_tc_tiling_on_sc=False),
        out_type=jax.ShapeDtypeStruct((B, LANES), jnp.float32),
        scratch_types=[
            pltpu.VMEM((ROWS, ROWLEN), jnp.float32),
            pltpu.VMEM((ROWS, ROWLEN), jnp.float32),
            pltpu.VMEM((CAP, ROWLEN), jnp.float32),
            pltpu.VMEM((LANES,), jnp.float32),
            pltpu.SemaphoreType.DMA,
            pltpu.SemaphoreType.DMA,
        ],
    )(_sc_body)
    return f(cp, ct)


def _tc_reduce_body(part_ref, sp_ref, st_ref, out_ref):
    p = part_ref[...]
    lane = lax.broadcasted_iota(jnp.int32, p.shape, 1)
    sq_sum = jnp.sum(jnp.where(lane == 0, p, 0.0))
    n_keep = jnp.sum(jnp.where(lane == 1, p, 0.0))
    court = sq_sum / jnp.maximum(n_keep, 1.0)
    d = sp_ref[...] - st_ref[...]
    score = jnp.sum(d * d) / float(B * 8)
    out_ref[0, 0] = court + score


def _tc_reduce(partials, sp, st):
    return pl.pallas_call(
        _tc_reduce_body,
        out_shape=jax.ShapeDtypeStruct((1, 1), jnp.float32),
        out_specs=pl.BlockSpec(memory_space=pltpu.SMEM),
    )(partials, sp, st)


def kernel(court_preds, score_preds, court_targs, score_targs):
    cp = court_preds.reshape(B, ROWS, ROWLEN)
    ct = court_targs.reshape(B, ROWS, ROWLEN)
    partials = _sc_partials(cp, ct)
    out = _tc_reduce(partials, score_preds, score_targs)
    return out[0, 0]


# speedup vs baseline: 1.7482x; 1.0020x over previous
import functools

import numpy as np
import jax
import jax.numpy as jnp
from jax import lax
from jax.experimental import pallas as pl
from jax.experimental.pallas import tpu as pltpu
from jax.experimental.pallas import tpu_sc as plsc

B = 32
N = 224 * 224
LANES = 16
ROWLEN = 128
ROWS = N // ROWLEN
SUB = ROWLEN // LANES
MIN_I32 = -2147483648
CAP = 224


def _sortable(v):
    return v ^ ((v >> 31) & 0x7FFFFFFF)


def _sc_body(cp_hbm, ct_hbm, out_hbm, a_ref, b_ref, c_ref, res_ref,
             sem1, sem2):
    wid = lax.axis_index("s") * 2 + lax.axis_index("c")
    h1 = pltpu.async_copy(cp_hbm.at[wid], a_ref, sem1)
    h2 = pltpu.async_copy(ct_hbm.at[wid], b_ref, sem2)
    h1.wait()
    h2.wait()

    iota16 = lax.iota(jnp.int32, LANES)
    zero_i = jnp.zeros((LANES,), jnp.int32)

    def p0(i, carry):
        npos_vec, nneg_vec = carry
        for u in range(SUB):
            ctv = b_ref[i, pl.ds(u * LANES, LANES)]
            v = plsc.bitcast(a_ref[i, pl.ds(u * LANES, LANES)], jnp.int32)
            a_ref[i, pl.ds(u * LANES, LANES)] = plsc.bitcast(
                _sortable(v), jnp.float32)
            npos_vec = npos_vec + jnp.where(ctv > 0.1, 1, 0)
            nneg_vec = nneg_vec + jnp.where(v >= 0, 1, 0)
        return npos_vec, nneg_vec

    npos_vec, nneg_vec = lax.fori_loop(0, ROWS, p0, (zero_i, zero_i))
    num_pos = jnp.sum(npos_vec)
    k = jnp.minimum(3 * num_pos, N - 1)
    cnt31 = jnp.sum(nneg_vec)

    def count_ge(cand_signed):
        def body(i, cnt_vec):
            for u in range(SUB):
                s = plsc.bitcast(a_ref[i, pl.ds(u * LANES, LANES)], jnp.int32)
                cnt_vec = cnt_vec + jnp.where(s >= cand_signed, 1, 0)
            return cnt_vec
        return jnp.sum(lax.fori_loop(0, ROWS, body, zero_i))

    def sbody(it, t_off):
        cand = t_off | jnp.left_shift(1, 31 - it)
        cnt = count_ge(cand ^ MIN_I32)
        return jnp.where(cnt >= k, cand, t_off)

    t31 = jnp.where(cnt31 >= k, np.int32(MIN_I32), np.int32(0))
    t_off8 = lax.fori_loop(1, 8, sbody, t31)
    p8 = lax.shift_right_logical(t_off8, 24)

    def compact(i, carry):
        cnts = carry[:SUB]
        above_vec = carry[SUB]
        new_cnts = []
        for u in range(SUB):
            cnt_u = cnts[u]
            sv = a_ref[i, pl.ds(u * LANES, LANES)]
            s = plsc.bitcast(sv, jnp.int32)
            bf = lax.shift_right_logical(s ^ MIN_I32, 24)
            match = bf == p8
            row = jnp.minimum(cnt_u, CAP - 1)
            col = iota16 + u * LANES
            plsc.store_scatter(c_ref, [row, col], sv,
                               mask=match & (cnt_u < CAP))
            new_cnts.append(cnt_u + jnp.where(match, 1, 0))
            above_vec = above_vec + jnp.where(bf > p8, 1, 0)
        return tuple(new_cnts) + (above_vec,)

    init = tuple(zero_i for _ in range(SUB)) + (zero_i,)
    res_c = lax.fori_loop(0, ROWS, compact, init)
    cnts = res_c[:SUB]
    above_cls = jnp.sum(res_c[SUB])
    mx = cnts[0]
    for u in range(1, SUB):
        mx = jnp.maximum(mx, cnts[u])
    maxc = jnp.max(mx)

    def finish_compacted(_):
        def count_ge_c(cand_signed):
            def body(rr, cnt_vec):
                for u in range(SUB):
                    sv = c_ref[rr, pl.ds(u * LANES, LANES)]
                    s = plsc.bitcast(sv, jnp.int32)
                    keep = (rr < cnts[u]) & (s >= cand_signed)
                    cnt_vec = cnt_vec + jnp.where(keep, 1, 0)
                return cnt_vec
            return jnp.sum(lax.fori_loop(0, maxc, body, zero_i))

        def sbody_c(it, t_off):
            cand = t_off | jnp.left_shift(1, 23 - it)
            cnt = above_cls + count_ge_c(cand ^ MIN_I32)
            return jnp.where(cnt >= k, cand, t_off)

        t_off = lax.fori_loop(0, 24, sbody_c, t_off8)
        tt = t_off ^ MIN_I32

        def cpass(rr, carry):
            gt_vec, ge_vec = carry
            for u in range(SUB):
                s = plsc.bitcast(c_ref[rr, pl.ds(u * LANES, LANES)],
                                 jnp.int32)
                valid = rr < cnts[u]
                gt_vec = gt_vec + jnp.where(valid & (s > tt), 1, 0)
                ge_vec = ge_vec + jnp.where(valid & (s >= tt), 1, 0)
            return gt_vec, ge_vec

        gt_vec, ge_vec = lax.fori_loop(0, maxc, cpass, (zero_i, zero_i))
        return tt, above_cls + jnp.sum(gt_vec), above_cls + jnp.sum(ge_vec)

    def finish_full(_):
        def sbody_f(it, t_off):
            cand = t_off | jnp.left_shift(1, 23 - it)
            cnt = count_ge(cand ^ MIN_I32)
            return jnp.where(cnt >= k, cand, t_off)

        t_off = lax.fori_loop(0, 24, sbody_f, t_off8)
        tt = t_off ^ MIN_I32

        def cpass(i, carry):
            gt_vec, ge_vec = carry
            for u in range(SUB):
                s = plsc.bitcast(a_ref[i, pl.ds(u * LANES, LANES)], jnp.int32)
                gt_vec = gt_vec + jnp.where(s > tt, 1, 0)
                ge_vec = ge_vec + jnp.where(s >= tt, 1, 0)
            return gt_vec, ge_vec

        gt_vec, ge_vec = lax.fori_loop(0, ROWS, cpass, (zero_i, zero_i))
        return tt, jnp.sum(gt_vec), jnp.sum(ge_vec)

    t, cnt_gt, cnt_ge = lax.cond(maxc <= CAP, finish_compacted, finish_full, 0)
    m = k - cnt_gt
    n_ties = cnt_ge - cnt_gt

    def idx_search(_):
        def count_tie_lt(cand):
            def body(i, cnt_vec):
                for u in range(SUB):
                    s = plsc.bitcast(
                        a_ref[i, pl.ds(u * LANES, LANES)], jnp.int32)
                    idx = i * ROWLEN + u * LANES + iota16
                    cnt_vec = cnt_vec + jnp.where((s == t) & (idx < cand), 1, 0)
                return cnt_vec
            return jnp.sum(lax.fori_loop(0, ROWS, body, zero_i))

        def ibody(it, cut):
            cand = cut | jnp.left_shift(1, 16 - it)
            return jnp.where(count_tie_lt(cand) <= m, cand, cut)

        return lax.fori_loop(0, 17, ibody, np.int32(0))

    cut = lax.cond(n_ties == m, lambda _: np.int32(131072), idx_search, 0)

    def fpass_simple(_):
        def fpass(i, carry):
            sq_vec, keep_vec = carry
            for u in range(SUB):
                s = plsc.bitcast(a_ref[i, pl.ds(u * LANES, LANES)], jnp.int32)
                cpv = plsc.bitcast(_sortable(s), jnp.float32)
                ctv = b_ref[i, pl.ds(u * LANES, LANES)]
                keep = (ctv > 0.1) | (s >= t)
                d = cpv - ctv
                sq_vec = sq_vec + jnp.where(keep, d * d, 0.0)
                keep_vec = keep_vec + jnp.where(keep, 1, 0)
            return sq_vec, keep_vec
        return lax.fori_loop(
            0, ROWS, fpass, (jnp.zeros((LANES,), jnp.float32), zero_i))

    def fpass_ties(_):
        def fpass(i, carry):
            sq_vec, keep_vec = carry
            for u in range(SUB):
                s = plsc.bitcast(a_ref[i, pl.ds(u * LANES, LANES)], jnp.int32)
                cpv = plsc.bitcast(_sortable(s), jnp.float32)
                ctv = b_ref[i, pl.ds(u * LANES, LANES)]
                idx = i * ROWLEN + u * LANES + iota16
                keep = (ctv > 0.1) | (s > t) | ((s == t) & (idx < cut))
                d = cpv - ctv
                sq_vec = sq_vec + jnp.where(keep, d * d, 0.0)
                keep_vec = keep_vec + jnp.where(keep, 1, 0)
            return sq_vec, keep_vec
        return lax.fori_loop(
            0, ROWS, fpass, (jnp.zeros((LANES,), jnp.float32), zero_i))

    sq_vec, keep_vec = lax.cond(cut == 131072, fpass_simple, fpass_ties, 0)
    sq_sum = jnp.sum(sq_vec)
    n_keep = jnp.sum(keep_vec).astype(jnp.float32)

    res = jnp.where(iota16 == 0, sq_sum,
                    jnp.where(iota16 == 1, n_keep, 0.0))
    res_ref[...] = res
    pltpu.sync_copy(res_ref, out_hbm.at[wid])


@functools.partial(jax.jit, static_argnums=())
def _sc_partials(cp, ct):
    mesh = plsc.VectorSubcoreMesh(core_axis_name="c", subcore_axis_name="s")
    f = functools.partial(
        pl.kernel,
        mesh=mesh,
        compiler_params=pltpu.CompilerParams(
            needs_layout_passes=False, use_tc_tiling_on_sc=False),
        out_type=jax.ShapeDtypeStruct((B, LANES), jnp.float32),
        scratch_types=[
            pltpu.VMEM((ROWS, ROWLEN), jnp.float32),
            pltpu.VMEM((ROWS, ROWLEN), jnp.float32),
            pltpu.VMEM((CAP, ROWLEN), jnp.float32),
            pltpu.VMEM((LANES,), jnp.float32),
            pltpu.SemaphoreType.DMA,
            pltpu.SemaphoreType.DMA,
        ],
    )(_sc_body)
    return f(cp, ct)


def _tc_reduce_body(part_ref, sp_ref, st_ref, out_ref):
    p = part_ref[...]
    lane = lax.broadcasted_iota(jnp.int32, p.shape, 1)
    sq_sum = jnp.sum(jnp.where(lane == 0, p, 0.0))
    n_keep = jnp.sum(jnp.where(lane == 1, p, 0.0))
    court = sq_sum / jnp.maximum(n_keep, 1.0)
    d = sp_ref[...] - st_ref[...]
    score = jnp.sum(d * d) / float(B * 8)
    out_ref[0, 0] = court + score


def _tc_reduce(partials, sp, st):
    return pl.pallas_call(
        _tc_reduce_body,
        out_shape=jax.ShapeDtypeStruct((1, 1), jnp.float32),
        out_specs=pl.BlockSpec(memory_space=pltpu.SMEM),
    )(partials, sp, st)


def kernel(court_preds, score_preds, court_targs, score_targs):
    cp = court_preds.reshape(B, ROWS, ROWLEN)
    ct = court_targs.reshape(B, ROWS, ROWLEN)
    partials = _sc_partials(cp, ct)
    out = _tc_reduce(partials, score_preds, score_targs)
    return out[0, 0]
